# baseline (speedup 1.0000x reference)
import functools

import jax
import jax.numpy as jnp
from jax import lax
from jax.experimental import pallas as pl
from jax.experimental.pallas import tpu as pltpu
from jax.experimental.pallas import tpu_sc as plsc

NC = 2
NS = 16
LANES = 16

NCH = 5
SCAT = 80


def _tc_transform(x, Wm, Wr, b2):
    N, DIN = x.shape
    RD = Wm.shape[1]
    DOUT = Wr.shape[1]
    R = RD // DOUT
    BN = 2000
    assert N % BN == 0

    def body(x_ref, wm_ref, wr_ref, b_ref, y_ref, r_ref):
        xb = x_ref[...]
        y = jnp.dot(xb, wm_ref[...], preferred_element_type=jnp.float32)
        for r in range(R):
            y_ref[r] = y[:, r * DOUT:(r + 1) * DOUT]
        r_ref[0] = (
            jnp.dot(xb, wr_ref[...], preferred_element_type=jnp.float32)
            + b_ref[...]
        )
        r_ref[1] = jnp.zeros((BN, DOUT), jnp.float32)

    return pl.pallas_call(
        body,
        grid=(N // BN,),
        in_specs=[
            pl.BlockSpec((BN, DIN), lambda i: (i, 0)),
            pl.BlockSpec((DIN, RD), lambda i: (0, 0)),
            pl.BlockSpec((DIN, DOUT), lambda i: (0, 0)),
            pl.BlockSpec((1, DOUT), lambda i: (0, 0)),
        ],
        out_specs=[
            pl.BlockSpec((R, BN, DOUT), lambda i: (0, i, 0)),
            pl.BlockSpec((2, BN, DOUT), lambda i: (0, i, 0)),
        ],
        out_shape=[
            jax.ShapeDtypeStruct((R, N, DOUT), jnp.float32),
            jax.ShapeDtypeStruct((2, N, DOUT), jnp.float32),
        ],
    )(x, Wm, Wr, b2)


def _make_sc_scatter(DOUT, EW, N_acc, N):
    CH = EW // NCH
    NSC = CH // SCAT
    RPT = N_acc // NS
    mesh = plsc.VectorSubcoreMesh(core_axis_name="c", subcore_axis_name="s")

    @functools.partial(
        pl.kernel,
        out_type=jax.ShapeDtypeStruct((NC, N_acc, DOUT), jnp.float32),
        mesh=mesh,
        compiler_params=pltpu.CompilerParams(use_tc_tiling_on_sc=False),
        scratch_types=[
            pltpu.VMEM((EW,), jnp.int32),
            pltpu.VMEM((EW,), jnp.int32),
            pltpu.VMEM((EW,), jnp.int32),
            pltpu.VMEM((CH, DOUT), jnp.float32),
            pltpu.VMEM((CH, DOUT), jnp.float32),
            pltpu.VMEM_SHARED((N_acc, DOUT), jnp.float32),
            pltpu.SemaphoreType.DMA,
            pltpu.SemaphoreType.DMA,
            pltpu.SemaphoreType.DMA,
        ],
    )
    def sc_kernel(table_hbm, src_hbm, type_hbm, dst_hbm, init_hbm, out_hbm,
                  sbuf, tbuf, dbuf, rows_a, rows_b, acc, sem_a, sem_b, sem_s):
        cid = lax.axis_index("c")
        sid = lax.axis_index("s")
        wid = sid * NC + cid
        base = wid * EW

        r0 = sid * RPT
        pltpu.sync_copy(init_hbm.at[cid].at[pl.ds(r0, RPT)],
                        acc.at[pl.ds(r0, RPT)])

        pltpu.sync_copy(src_hbm.at[pl.ds(base, EW)], sbuf)
        pltpu.sync_copy(type_hbm.at[pl.ds(base, EW)], tbuf)
        pltpu.sync_copy(dst_hbm.at[pl.ds(base, EW)], dbuf)

        def idx_body(i, _):
            s = sbuf[pl.ds(i * LANES, LANES)]
            t = tbuf[pl.ds(i * LANES, LANES)]
            sbuf[pl.ds(i * LANES, LANES)] = t * N + s
            return 0

        lax.fori_loop(0, EW // LANES, idx_body, 0)

        plsc.subcore_barrier()

        bufs = (rows_a, rows_b)
        sems = (sem_a, sem_b)
        cps = [None, None]
        cps[0] = pltpu.async_copy(
            table_hbm.at[sbuf.at[pl.ds(0, CH)]], rows_a, sem_a)
        for h in range(NCH):
            cur = bufs[h % 2]
            if h + 1 < NCH:
                cps[(h + 1) % 2] = pltpu.async_copy(
                    table_hbm.at[sbuf.at[pl.ds((h + 1) * CH, CH)]],
                    bufs[(h + 1) % 2], sems[(h + 1) % 2])
            cps[h % 2].wait()
            scs = []
            for j in range(NSC):
                scs.append(pltpu.async_copy(
                    cur.at[pl.ds(j * SCAT, SCAT)],
                    acc.at[dbuf.at[pl.ds(h * CH + j * SCAT, SCAT)]],
                    sem_s, add=True))
            for cp in scs:
                cp.wait()

        plsc.subcore_barrier()

        r0 = sid * RPT
        pltpu.sync_copy(acc.at[pl.ds(r0, RPT)],
                        out_hbm.at[cid].at[pl.ds(r0, RPT)])

    return sc_kernel


def kernel(x, edge_index, edge_type, edge_ptr, W, W_root, bias):
    N, DIN = x.shape
    R, _, DOUT = W.shape
    E = edge_type.shape[0]
    NW = NC * NS

    Wm = jnp.transpose(W, (1, 0, 2)).reshape(DIN, R * DOUT)
    table, init = _tc_transform(x, Wm, W_root, bias.reshape(1, DOUT))
    table = table.reshape(R * N, DOUT)

    assert E % (NW * NCH * SCAT) == 0
    EW = E // NW

    N_acc = ((N + NS - 1) // NS) * NS
    assert N_acc == N

    sc = _make_sc_scatter(DOUT, EW, N_acc, N)
    parts = sc(table, edge_index[0], edge_type, edge_index[1], init)
    return parts[0] + parts[1]


# speedup vs baseline: 1.4160x; 1.4160x over previous
import functools

import jax
import jax.numpy as jnp
from jax import lax
from jax.experimental import pallas as pl
from jax.experimental.pallas import tpu as pltpu
from jax.experimental.pallas import tpu_sc as plsc

NC = 2
NS = 16
LANES = 16

NCH = 5
SCAT = 80


def _tc_transform(x, Wm, Wr, b2):
    N, DIN = x.shape
    RD = Wm.shape[1]
    DOUT = Wr.shape[1]
    R = RD // DOUT
    BN = 2000
    assert N % BN == 0

    def body(x_ref, wm_ref, wr_ref, b_ref, y_ref, r_ref):
        xb = x_ref[...]
        y_ref[...] = jnp.dot(xb, wm_ref[...], preferred_element_type=jnp.float32)
        r_ref[0] = (
            jnp.dot(xb, wr_ref[...], preferred_element_type=jnp.float32)
            + b_ref[...]
        )
        r_ref[1] = jnp.zeros((BN, DOUT), jnp.float32)

    return pl.pallas_call(
        body,
        grid=(N // BN,),
        in_specs=[
            pl.BlockSpec((BN, DIN), lambda i: (i, 0)),
            pl.BlockSpec((DIN, RD), lambda i: (0, 0)),
            pl.BlockSpec((DIN, DOUT), lambda i: (0, 0)),
            pl.BlockSpec((1, DOUT), lambda i: (0, 0)),
        ],
        out_specs=[
            pl.BlockSpec((BN, RD), lambda i: (i, 0)),
            pl.BlockSpec((2, BN, DOUT), lambda i: (0, i, 0)),
        ],
        out_shape=[
            jax.ShapeDtypeStruct((N, RD), jnp.float32),
            jax.ShapeDtypeStruct((2, N, DOUT), jnp.float32),
        ],
    )(x, Wm, Wr, b2)


def _make_sc_scatter(DOUT, EW, N_acc, R):
    CH = EW // NCH
    NSC = CH // SCAT
    RPT = N_acc // NS
    mesh = plsc.VectorSubcoreMesh(core_axis_name="c", subcore_axis_name="s")

    @functools.partial(
        pl.kernel,
        out_type=jax.ShapeDtypeStruct((NC, N_acc, DOUT), jnp.float32),
        mesh=mesh,
        compiler_params=pltpu.CompilerParams(use_tc_tiling_on_sc=False),
        scratch_types=[
            pltpu.VMEM((EW,), jnp.int32),
            pltpu.VMEM((EW,), jnp.int32),
            pltpu.VMEM((EW,), jnp.int32),
            pltpu.VMEM((CH, DOUT), jnp.float32),
            pltpu.VMEM((CH, DOUT), jnp.float32),
            pltpu.VMEM_SHARED((N_acc, DOUT), jnp.float32),
            pltpu.SemaphoreType.DMA,
            pltpu.SemaphoreType.DMA,
            pltpu.SemaphoreType.DMA,
        ],
    )
    def sc_kernel(table_hbm, src_hbm, type_hbm, dst_hbm, init_hbm, out_hbm,
                  sbuf, tbuf, dbuf, rows_a, rows_b, acc, sem_a, sem_b, sem_s):
        cid = lax.axis_index("c")
        sid = lax.axis_index("s")
        wid = sid * NC + cid
        base = wid * EW

        r0 = sid * RPT
        pltpu.sync_copy(init_hbm.at[cid].at[pl.ds(r0, RPT)],
                        acc.at[pl.ds(r0, RPT)])

        pltpu.sync_copy(src_hbm.at[pl.ds(base, EW)], sbuf)
        pltpu.sync_copy(type_hbm.at[pl.ds(base, EW)], tbuf)
        pltpu.sync_copy(dst_hbm.at[pl.ds(base, EW)], dbuf)

        def idx_body(i, _):
            s = sbuf[pl.ds(i * LANES, LANES)]
            t = tbuf[pl.ds(i * LANES, LANES)]
            sbuf[pl.ds(i * LANES, LANES)] = s * R + t
            return 0

        lax.fori_loop(0, EW // LANES, idx_body, 0)

        plsc.subcore_barrier()

        bufs = (rows_a, rows_b)
        sems = (sem_a, sem_b)
        cps = [None, None]
        cps[0] = pltpu.async_copy(
            table_hbm.at[sbuf.at[pl.ds(0, CH)]], rows_a, sem_a)
        for h in range(NCH):
            cur = bufs[h % 2]
            if h + 1 < NCH:
                cps[(h + 1) % 2] = pltpu.async_copy(
                    table_hbm.at[sbuf.at[pl.ds((h + 1) * CH, CH)]],
                    bufs[(h + 1) % 2], sems[(h + 1) % 2])
            cps[h % 2].wait()
            scs = []
            for j in range(NSC):
                scs.append(pltpu.async_copy(
                    cur.at[pl.ds(j * SCAT, SCAT)],
                    acc.at[dbuf.at[pl.ds(h * CH + j * SCAT, SCAT)]],
                    sem_s, add=True))
            for cp in scs:
                cp.wait()

        plsc.subcore_barrier()

        r0 = sid * RPT
        pltpu.sync_copy(acc.at[pl.ds(r0, RPT)],
                        out_hbm.at[cid].at[pl.ds(r0, RPT)])

    return sc_kernel


def kernel(x, edge_index, edge_type, edge_ptr, W, W_root, bias):
    N, DIN = x.shape
    R, _, DOUT = W.shape
    E = edge_type.shape[0]
    NW = NC * NS

    Wm = jnp.transpose(W, (1, 0, 2)).reshape(DIN, R * DOUT)
    y_msg, init = _tc_transform(x, Wm, W_root, bias.reshape(1, DOUT))
    table = y_msg.reshape(N * R, DOUT)

    assert E % (NW * NCH * SCAT) == 0
    EW = E // NW

    N_acc = ((N + NS - 1) // NS) * NS
    assert N_acc == N

    sc = _make_sc_scatter(DOUT, EW, N_acc, R)
    parts = sc(table, edge_index[0], edge_type, edge_index[1], init)
    return parts[0] + parts[1]
